# Initial kernel scaffold; baseline (speedup 1.0000x reference)
import jax
import jax.numpy as jnp
from jax.experimental import pallas as pl


def kernel(images):
    raise NotImplementedError("write your pallas kernel here")


# speedup vs baseline: 18.4182x; 18.4182x over previous
import functools

import jax
import jax.numpy as jnp
from jax import lax
from jax.experimental import pallas as pl
from jax.experimental.pallas import tpu as pltpu
from jax.experimental.pallas import tpu_sc as plsc

NIMG = 64
HW = 512 * 512
IMG = HW * 3
N = NIMG * IMG
CH = 49152
NCHUNK = IMG // CH
OWPC = CH // 4
NW = 32
IPW = NIMG // NW


def _equalize_body(img_hbm, out_hbm, buf, obuf, hist, lut, csbuf):
    wid = lax.axis_index("s") * 2 + lax.axis_index("c")
    iota = lax.iota(jnp.int32, 16)
    ones = jnp.full((16,), 1, jnp.int32)
    zeros = jnp.zeros((16,), jnp.int32)
    hconst = [iota * 768 + ((iota + r) % 3) * 256 for r in range(3)]
    pconst = [4 * iota + c for c in range(4)]
    chof3 = [((iota + m) % 3) * 256 for m in range(3)]

    for img_slot in range(IPW):
        img = wid * IPW + img_slot
        ibase = img * IMG
        obase = img * (IMG // 4)

        def zbody(j, _):
            hist[pl.ds(j * 16, 16)] = zeros
            return 0

        lax.fori_loop(0, 768, zbody, 0)

        def hchunk(ci, _):
            off = pl.multiple_of(ibase + ci * CH, 8)
            pltpu.sync_copy(img_hbm.at[pl.ds(off, CH)], buf)

            def hbody(t, _):
                for r in range(3):
                    v = buf[pl.ds((t * 3 + r) * 16, 16)]
                    plsc.addupdate_scatter(hist, [hconst[r] + v], ones)
                return 0

            lax.fori_loop(0, CH // 48, hbody, 0)
            return 0

        lax.fori_loop(0, NCHUNK, hchunk, 0)

        for c in range(3):
            cbase = c * 256

            def lane_red(k):
                def lbody(l, a):
                    return a + hist[pl.ds(l * 768 + cbase + k * 16, 16)]

                return lax.fori_loop(1, 16, lbody, hist[pl.ds(cbase + k * 16, 16)])

            def csbody(k, carry):
                run_s, maxcode = carry
                acc = lane_red(k)
                cs = plsc.cumsum(acc) + run_s
                csbuf[pl.ds(k * 16, 16)] = cs
                binv = iota + k * 16
                code = jnp.where(acc > 0, binv * 524288 + acc, -1)
                return jnp.max(cs), jnp.maximum(maxcode, code)

            run_s, maxcode = lax.fori_loop(
                0, 16, csbody, (jnp.int32(0), jnp.full((16,), -1, jnp.int32))
            )

            last_val = jnp.max(maxcode) & 524287
            step = lax.div(jnp.int32(HW) - last_val, jnp.int32(255))
            safe = jnp.broadcast_to(jnp.maximum(step, 1), (16,))
            half = lax.div(step, jnp.int32(2))
            is_id = jnp.broadcast_to(step == 0, (16,))

            def lutbody(k, _):
                binv = iota + k * 16
                sidx = jnp.maximum(binv - 1, 0)
                prevcs = plsc.load_gather(csbuf, [sidx])
                prevcs = jnp.where(binv == 0, 0, prevcs)
                lv = jnp.clip(lax.div(prevcs + half, safe), 0, 255)
                lv = jnp.where(is_id, binv, lv)
                lut[pl.ds(cbase + k * 16, 16)] = lv
                return 0

            lax.fori_loop(0, 16, lutbody, 0)

        def mchunk(ci, _):
            off = pl.multiple_of(ibase + ci * CH, 8)
            pltpu.sync_copy(img_hbm.at[pl.ds(off, CH)], buf)

            def mbody(t, _):
                for r in range(3):
                    j = t * 3 + r
                    jb = j * 64
                    ws = []
                    for c in range(4):
                        pix = plsc.load_gather(buf, [pconst[c] + jb])
                        mv = plsc.load_gather(lut, [pix + chof3[(r + c) % 3]])
                        ws.append(mv)
                    w = ws[0] | (ws[1] << 8) | (ws[2] << 16) | (ws[3] << 24)
                    obuf[pl.ds(j * 16, 16)] = w
                return 0

            lax.fori_loop(0, CH // 192, mbody, 0)
            ooff = pl.multiple_of(obase + ci * OWPC, 8)
            pltpu.sync_copy(obuf, out_hbm.at[pl.ds(ooff, OWPC)])
            return 0

        lax.fori_loop(0, NCHUNK, mchunk, 0)


_equalize = functools.partial(
    pl.kernel,
    out_type=jax.ShapeDtypeStruct((N // 4,), jnp.int32),
    mesh=plsc.VectorSubcoreMesh(core_axis_name="c", subcore_axis_name="s"),
    compiler_params=pltpu.CompilerParams(needs_layout_passes=False),
    scratch_types=[
        pltpu.VMEM((CH,), jnp.int32),
        pltpu.VMEM((OWPC,), jnp.int32),
        pltpu.VMEM((16 * 768,), jnp.int32),
        pltpu.VMEM((768,), jnp.int32),
        pltpu.VMEM((256,), jnp.int32),
    ],
)(_equalize_body)


@jax.jit
def kernel(images):
    flat = images.astype(jnp.int32).reshape(-1)
    out32 = _equalize(flat)
    out8 = lax.bitcast_convert_type(out32, jnp.uint8)
    return out8.reshape(images.shape)
